# baseline (speedup 1.0000x reference)
import functools

import jax
import jax.numpy as jnp
from jax.experimental import pallas as pl
from jax.experimental.pallas import tpu as pltpu

_CDT = jnp.bfloat16


def _shift_rows(a, s, fill=0.0):
    if s == 0:
        return a
    pad = jnp.full((abs(s), a.shape[1]), fill, a.dtype)
    if s > 0:
        return jnp.concatenate([a[s:], pad], axis=0)
    return jnp.concatenate([pad, a[:s]], axis=0)


def _shift_cols(a, s, fill=0.0):
    if s == 0:
        return a
    pad = jnp.full((a.shape[0], abs(s)), fill, a.dtype)
    if s > 0:
        return jnp.concatenate([a[:, s:], pad], axis=1)
    return jnp.concatenate([pad, a[:, :s]], axis=1)


def _conv3x3x3(t, w, b, masks, W, H, channel_major):
    M, C = t.shape
    mw_lo, mw_hi, mh_lo, mh_hi = masks
    tw = jnp.concatenate([
        _shift_rows(t, -1) * mw_lo,
        t,
        _shift_rows(t, 1) * mw_hi,
    ], axis=1)
    th = jnp.concatenate([
        _shift_rows(tw, -W) * mh_lo,
        tw,
        _shift_rows(tw, W) * mh_hi,
    ], axis=1)
    K9 = 9 * C
    acc = None
    for kd in range(3):
        td = _shift_rows(th, (kd - 1) * W * H)
        wk = w[kd * K9:(kd + 1) * K9, :]
        if channel_major:
            p = jax.lax.dot_general(wk, td, (((0,), (1,)), ((), ())),
                                    preferred_element_type=jnp.float32)
        else:
            p = jax.lax.dot_general(td, wk, (((1,), (0,)), ((), ())),
                                    preferred_element_type=jnp.float32)
        acc = p if acc is None else acc + p
    return jnp.maximum(acc + b, 0.0)


def _maxpool3(t, masks, W, H):
    neg = float(jnp.finfo(jnp.float32).min)
    mw_lo, mw_hi, mh_lo, mh_hi = masks

    def tap3(a, s, mlo, mhi):
        lo = jnp.where(mlo, _shift_rows(a, -s, neg), neg).astype(a.dtype)
        hi = jnp.where(mhi, _shift_rows(a, s, neg), neg).astype(a.dtype)
        return jnp.maximum(jnp.maximum(lo, a), hi)

    p = tap3(t, 1, mw_lo, mw_hi)
    p = tap3(p, W, mh_lo, mh_hi)
    return jnp.maximum(jnp.maximum(_shift_rows(p, -W * H, neg), p),
                       _shift_rows(p, W * H, neg))


def _inception_kernel(x_ref, stem_w_ref, b1_ref, b2s_ref, b3s_ref,
                      w2_ref, b2_ref, w31_ref, b31_ref,
                      w32_ref, b32_ref, w4_ref, b4_ref, o_ref,
                      *, D, H, W, c1, c2, n2, n3):
    M = D * H * W
    w1 = stem_w_ref[:, :c1].astype(_CDT)
    w23 = stem_w_ref[:, c1:].astype(_CDT)
    b1c = b1_ref[...].T
    b23 = jnp.concatenate([b2s_ref[...], b3s_ref[...]], axis=1)
    w2 = w2_ref[...].astype(_CDT)
    b2c = b2_ref[...].T
    w31 = w31_ref[...].astype(_CDT)
    b31 = b31_ref[...]
    w32 = w32_ref[...].astype(_CDT)
    b32c = b32_ref[...].T
    w4 = w4_ref[...].astype(_CDT)
    b4c = b4_ref[...].T

    x_cl = x_ref[0].T.astype(_CDT)

    ri = jax.lax.broadcasted_iota(jnp.int32, (M, 1), 0)
    wc = ri % W
    hc = (ri // W) % H
    bmasks = (wc > 0, wc < W - 1, hc > 0, hc < H - 1)
    masks = tuple(m.astype(_CDT) for m in bmasks)

    y1 = jax.lax.dot_general(w1, x_cl, (((0,), (1,)), ((), ())),
                             preferred_element_type=jnp.float32)
    o_ref[0, :c1, :] = jnp.maximum(y1 + b1c, 0.0)

    t23 = jax.lax.dot_general(x_cl, w23, (((1,), (0,)), ((), ())),
                              preferred_element_type=jnp.float32)
    t23 = jnp.maximum(t23 + b23, 0.0)
    t2 = t23[:, :c2].astype(_CDT)
    t3 = t23[:, c2:].astype(_CDT)

    y2 = _conv3x3x3(t2, w2, b2c, masks, W, H, channel_major=True)
    o_ref[0, c1:c1 + n2, :] = y2

    t3b = _conv3x3x3(t3, w31, b31, masks, W, H,
                     channel_major=False).astype(_CDT)
    y3 = _conv3x3x3(t3b, w32, b32c, masks, W, H, channel_major=True)
    o_ref[0, c1 + n2:c1 + n2 + n3, :] = y3

    pooled = _maxpool3(x_cl, bmasks, W, H)
    y4 = jax.lax.dot_general(w4, pooled, (((0,), (1,)), ((), ())),
                             preferred_element_type=jnp.float32)
    o_ref[0, c1 + n2 + n3:, :] = jnp.maximum(y4 + b4c, 0.0)


def kernel(x, stem_w, stem_b1, stem_b2, stem_b3,
           b2_1_w, b2_1_b, b3_1_w, b3_1_b, b3_2_w, b3_2_b,
           b4_1_w, b4_1_b):
    N, Cin, D, H, W = x.shape
    M = D * H * W
    c1 = stem_b1.shape[1]
    c2 = stem_b2.shape[1]
    n2 = b2_1_w.shape[1]
    n3 = b3_2_w.shape[1]
    n4 = b4_1_w.shape[1]
    cout = c1 + n2 + n3 + n4

    xr = x.reshape(N, Cin, M)

    kfn = functools.partial(_inception_kernel, D=D, H=H, W=W,
                            c1=c1, c2=c2, n2=n2, n3=n3)

    def full(a):
        nd = len(a.shape)
        return pl.BlockSpec(a.shape, lambda n, _nd=nd: (0,) * _nd)

    out = pl.pallas_call(
        kfn,
        out_shape=jax.ShapeDtypeStruct((N, cout, M), jnp.float32),
        grid=(N,),
        in_specs=[
            pl.BlockSpec((1, Cin, M), lambda n: (n, 0, 0)),
            full(stem_w), full(stem_b1), full(stem_b2), full(stem_b3),
            full(b2_1_w), full(b2_1_b), full(b3_1_w), full(b3_1_b),
            full(b3_2_w), full(b3_2_b), full(b4_1_w), full(b4_1_b),
        ],
        out_specs=pl.BlockSpec((1, cout, M), lambda n: (n, 0, 0)),
        compiler_params=pltpu.CompilerParams(
            dimension_semantics=("parallel",)),
    )(xr, stem_w, stem_b1, stem_b2, stem_b3,
      b2_1_w, b2_1_b, b3_1_w, b3_1_b, b3_2_w, b3_2_b, b4_1_w, b4_1_b)
    return out.reshape(N, cout, D, H, W)


# speedup vs baseline: 1.0061x; 1.0061x over previous
import functools

import jax
import jax.numpy as jnp
from jax.experimental import pallas as pl
from jax.experimental.pallas import tpu as pltpu

_CDT = jnp.bfloat16


def _shift_rows(a, s, fill=0.0):
    if s == 0:
        return a
    pad = jnp.full((abs(s), a.shape[1]), fill, a.dtype)
    if s > 0:
        return jnp.concatenate([a[s:], pad], axis=0)
    return jnp.concatenate([pad, a[:s]], axis=0)


def _shift_cols(a, s, fill=0.0):
    if s == 0:
        return a
    pad = jnp.full((a.shape[0], abs(s)), fill, a.dtype)
    if s > 0:
        return jnp.concatenate([a[:, s:], pad], axis=1)
    return jnp.concatenate([pad, a[:, :s]], axis=1)


def _conv3x3x3(t, w, b, masks, W, H, channel_major):
    M, C = t.shape
    mw_lo, mw_hi, mh_lo, mh_hi = masks
    tw = jnp.concatenate([
        jnp.where(mw_lo, _shift_rows(t, -1), 0).astype(t.dtype),
        t,
        jnp.where(mw_hi, _shift_rows(t, 1), 0).astype(t.dtype),
    ], axis=1)
    th = jnp.concatenate([
        jnp.where(mh_lo, _shift_rows(tw, -W), 0).astype(t.dtype),
        tw,
        jnp.where(mh_hi, _shift_rows(tw, W), 0).astype(t.dtype),
    ], axis=1)
    K9 = 9 * C
    acc = None
    for kd in range(3):
        td = _shift_rows(th, (kd - 1) * W * H)
        wk = w[kd * K9:(kd + 1) * K9, :]
        if channel_major:
            p = jax.lax.dot_general(wk, td, (((0,), (1,)), ((), ())),
                                    preferred_element_type=jnp.float32)
        else:
            p = jax.lax.dot_general(td, wk, (((1,), (0,)), ((), ())),
                                    preferred_element_type=jnp.float32)
        acc = p if acc is None else acc + p
    return jnp.maximum(acc + b, 0.0)


def _maxpool3(t, masks, W, H):
    neg = float(jnp.finfo(jnp.float32).min)
    mw_lo, mw_hi, mh_lo, mh_hi = masks

    def tap3(a, s, mlo, mhi):
        lo = jnp.where(mlo, _shift_rows(a, -s, neg), neg).astype(a.dtype)
        hi = jnp.where(mhi, _shift_rows(a, s, neg), neg).astype(a.dtype)
        return jnp.maximum(jnp.maximum(lo, a), hi)

    p = tap3(t, 1, mw_lo, mw_hi)
    p = tap3(p, W, mh_lo, mh_hi)
    return jnp.maximum(jnp.maximum(_shift_rows(p, -W * H, neg), p),
                       _shift_rows(p, W * H, neg))


def _inception_kernel(x_ref, stem_w_ref, b1_ref, b2s_ref, b3s_ref,
                      w2_ref, b2_ref, w31_ref, b31_ref,
                      w32_ref, b32_ref, w4_ref, b4_ref, o_ref,
                      *, D, H, W, c1, c2, n2, n3):
    M = D * H * W
    w1 = stem_w_ref[:, :c1].astype(_CDT)
    w23 = stem_w_ref[:, c1:].astype(_CDT)
    b1c = b1_ref[...].T
    b23 = jnp.concatenate([b2s_ref[...], b3s_ref[...]], axis=1)
    w2 = w2_ref[...].astype(_CDT)
    b2c = b2_ref[...].T
    w31 = w31_ref[...].astype(_CDT)
    b31 = b31_ref[...]
    w32 = w32_ref[...].astype(_CDT)
    b32c = b32_ref[...].T
    w4 = w4_ref[...].astype(_CDT)
    b4c = b4_ref[...].T

    x_cl = x_ref[0].T.astype(_CDT)

    ri = jax.lax.broadcasted_iota(jnp.int32, (M, 1), 0)
    wc = ri % W
    hc = (ri // W) % H
    masks = (wc > 0, wc < W - 1, hc > 0, hc < H - 1)

    y1 = jax.lax.dot_general(w1, x_cl, (((0,), (1,)), ((), ())),
                             preferred_element_type=jnp.float32)
    o_ref[0, :c1, :] = jnp.maximum(y1 + b1c, 0.0)

    t23 = jax.lax.dot_general(x_cl, w23, (((1,), (0,)), ((), ())),
                              preferred_element_type=jnp.float32)
    t23 = jnp.maximum(t23 + b23, 0.0)
    t2 = t23[:, :c2].astype(_CDT)
    t3 = t23[:, c2:].astype(_CDT)

    y2 = _conv3x3x3(t2, w2, b2c, masks, W, H, channel_major=True)
    o_ref[0, c1:c1 + n2, :] = y2

    t3b = _conv3x3x3(t3, w31, b31, masks, W, H,
                     channel_major=False).astype(_CDT)
    y3 = _conv3x3x3(t3b, w32, b32c, masks, W, H, channel_major=True)
    o_ref[0, c1 + n2:c1 + n2 + n3, :] = y3

    pooled = _maxpool3(x_cl, masks, W, H)
    y4 = jax.lax.dot_general(w4, pooled, (((0,), (1,)), ((), ())),
                             preferred_element_type=jnp.float32)
    o_ref[0, c1 + n2 + n3:, :] = jnp.maximum(y4 + b4c, 0.0)


def kernel(x, stem_w, stem_b1, stem_b2, stem_b3,
           b2_1_w, b2_1_b, b3_1_w, b3_1_b, b3_2_w, b3_2_b,
           b4_1_w, b4_1_b):
    N, Cin, D, H, W = x.shape
    M = D * H * W
    c1 = stem_b1.shape[1]
    c2 = stem_b2.shape[1]
    n2 = b2_1_w.shape[1]
    n3 = b3_2_w.shape[1]
    n4 = b4_1_w.shape[1]
    cout = c1 + n2 + n3 + n4

    xr = x.reshape(N, Cin, M)

    kfn = functools.partial(_inception_kernel, D=D, H=H, W=W,
                            c1=c1, c2=c2, n2=n2, n3=n3)

    def full(a):
        nd = len(a.shape)
        return pl.BlockSpec(a.shape, lambda n, _nd=nd: (0,) * _nd)

    out = pl.pallas_call(
        kfn,
        out_shape=jax.ShapeDtypeStruct((N, cout, M), jnp.float32),
        grid=(N,),
        in_specs=[
            pl.BlockSpec((1, Cin, M), lambda n: (n, 0, 0)),
            full(stem_w), full(stem_b1), full(stem_b2), full(stem_b3),
            full(b2_1_w), full(b2_1_b), full(b3_1_w), full(b3_1_b),
            full(b3_2_w), full(b3_2_b), full(b4_1_w), full(b4_1_b),
        ],
        out_specs=pl.BlockSpec((1, cout, M), lambda n: (n, 0, 0)),
        compiler_params=pltpu.CompilerParams(
            dimension_semantics=("parallel",)),
    )(xr, stem_w, stem_b1, stem_b2, stem_b3,
      b2_1_w, b2_1_b, b3_1_w, b3_1_b, b3_2_w, b3_2_b, b4_1_w, b4_1_b)
    return out.reshape(N, cout, D, H, W)


# speedup vs baseline: 1.0532x; 1.0468x over previous
import functools

import jax
import jax.numpy as jnp
from jax.experimental import pallas as pl
from jax.experimental.pallas import tpu as pltpu

_CDT = jnp.bfloat16


def _shift_rows(a, s, fill=0.0):
    if s == 0:
        return a
    pad = jnp.full((abs(s), a.shape[1]), fill, a.dtype)
    if s > 0:
        return jnp.concatenate([a[s:], pad], axis=0)
    return jnp.concatenate([pad, a[:s]], axis=0)


def _shift_cols(a, s, fill=0.0):
    if s == 0:
        return a
    pad = jnp.full((a.shape[0], abs(s)), fill, a.dtype)
    if s > 0:
        return jnp.concatenate([a[:, s:], pad], axis=1)
    return jnp.concatenate([pad, a[:, :s]], axis=1)


def _conv3x3x3(t, w, b, masks, W, H, channel_major):
    M, C = t.shape
    mw_lo, mw_hi, mh_lo, mh_hi = masks
    tw = jnp.concatenate([
        jnp.where(mw_lo, _shift_rows(t, -1), 0).astype(t.dtype),
        t,
        jnp.where(mw_hi, _shift_rows(t, 1), 0).astype(t.dtype),
    ], axis=1)
    th = jnp.concatenate([
        jnp.where(mh_lo, _shift_rows(tw, -W), 0).astype(t.dtype),
        tw,
        jnp.where(mh_hi, _shift_rows(tw, W), 0).astype(t.dtype),
    ], axis=1)
    K9 = 9 * C
    acc = None
    for kd in range(3):
        td = _shift_rows(th, (kd - 1) * W * H)
        wk = w[kd * K9:(kd + 1) * K9, :]
        if channel_major:
            p = jax.lax.dot_general(wk, td, (((0,), (1,)), ((), ())),
                                    preferred_element_type=jnp.float32)
        else:
            p = jax.lax.dot_general(td, wk, (((1,), (0,)), ((), ())),
                                    preferred_element_type=jnp.float32)
        acc = p if acc is None else acc + p
    return jnp.maximum(acc + b, 0.0)


def _maxpool3(t, masks, W, H):
    neg = float(jnp.finfo(jnp.float32).min)
    mw_lo, mw_hi, mh_lo, mh_hi = masks

    def tap3(a, s, mlo, mhi):
        lo = jnp.where(mlo, _shift_rows(a, -s, neg), neg).astype(a.dtype)
        hi = jnp.where(mhi, _shift_rows(a, s, neg), neg).astype(a.dtype)
        return jnp.maximum(jnp.maximum(lo, a), hi)

    p = tap3(t, 1, mw_lo, mw_hi)
    p = tap3(p, W, mh_lo, mh_hi)
    return jnp.maximum(jnp.maximum(_shift_rows(p, -W * H, neg), p),
                       _shift_rows(p, W * H, neg))


def _inception_kernel(x_ref, stem_w_ref, b1_ref, b2s_ref, b3s_ref,
                      w2_ref, b2_ref, w31_ref, b31_ref,
                      w32_ref, b32_ref, w4_ref, b4_ref, o_ref,
                      *, D, H, W, c1, c2, n2, n3):
    M = D * H * W
    w1 = stem_w_ref[:, :c1].astype(_CDT)
    w23 = stem_w_ref[:, c1:].astype(_CDT)
    b1c = b1_ref[...].T
    b23 = jnp.concatenate([b2s_ref[...], b3s_ref[...]], axis=1)
    w2 = w2_ref[...].astype(_CDT)
    b2c = b2_ref[...].T
    w31 = w31_ref[...].astype(_CDT)
    b31 = b31_ref[...]
    w32 = w32_ref[...].astype(_CDT)
    b32c = b32_ref[...].T
    w4 = w4_ref[...].astype(_CDT)
    b4c = b4_ref[...].T

    x_cl = x_ref[0].T.astype(_CDT)

    rhw = jax.lax.broadcasted_iota(jnp.int32, (H * W, 1), 0)
    wc = rhw % W
    hc = rhw // W
    masks = tuple(jnp.concatenate([m] * D, axis=0)
                  for m in (wc > 0, wc < W - 1, hc > 0, hc < H - 1))

    y1 = jax.lax.dot_general(w1, x_cl, (((0,), (1,)), ((), ())),
                             preferred_element_type=jnp.float32)
    o_ref[0, :c1, :] = jnp.maximum(y1 + b1c, 0.0)

    t23 = jax.lax.dot_general(x_cl, w23, (((1,), (0,)), ((), ())),
                              preferred_element_type=jnp.float32)
    t23 = jnp.maximum(t23 + b23, 0.0)
    t2 = t23[:, :c2].astype(_CDT)
    t3 = t23[:, c2:].astype(_CDT)

    y2 = _conv3x3x3(t2, w2, b2c, masks, W, H, channel_major=True)
    o_ref[0, c1:c1 + n2, :] = y2

    t3b = _conv3x3x3(t3, w31, b31, masks, W, H,
                     channel_major=False).astype(_CDT)
    y3 = _conv3x3x3(t3b, w32, b32c, masks, W, H, channel_major=True)
    o_ref[0, c1 + n2:c1 + n2 + n3, :] = y3

    pooled = _maxpool3(x_cl, masks, W, H)
    y4 = jax.lax.dot_general(w4, pooled, (((0,), (1,)), ((), ())),
                             preferred_element_type=jnp.float32)
    o_ref[0, c1 + n2 + n3:, :] = jnp.maximum(y4 + b4c, 0.0)


def kernel(x, stem_w, stem_b1, stem_b2, stem_b3,
           b2_1_w, b2_1_b, b3_1_w, b3_1_b, b3_2_w, b3_2_b,
           b4_1_w, b4_1_b):
    N, Cin, D, H, W = x.shape
    M = D * H * W
    c1 = stem_b1.shape[1]
    c2 = stem_b2.shape[1]
    n2 = b2_1_w.shape[1]
    n3 = b3_2_w.shape[1]
    n4 = b4_1_w.shape[1]
    cout = c1 + n2 + n3 + n4

    xr = x.reshape(N, Cin, M)

    kfn = functools.partial(_inception_kernel, D=D, H=H, W=W,
                            c1=c1, c2=c2, n2=n2, n3=n3)

    def full(a):
        nd = len(a.shape)
        return pl.BlockSpec(a.shape, lambda n, _nd=nd: (0,) * _nd)

    out = pl.pallas_call(
        kfn,
        out_shape=jax.ShapeDtypeStruct((N, cout, M), jnp.float32),
        grid=(N,),
        in_specs=[
            pl.BlockSpec((1, Cin, M), lambda n: (n, 0, 0)),
            full(stem_w), full(stem_b1), full(stem_b2), full(stem_b3),
            full(b2_1_w), full(b2_1_b), full(b3_1_w), full(b3_1_b),
            full(b3_2_w), full(b3_2_b), full(b4_1_w), full(b4_1_b),
        ],
        out_specs=pl.BlockSpec((1, cout, M), lambda n: (n, 0, 0)),
        compiler_params=pltpu.CompilerParams(
            dimension_semantics=("parallel",)),
    )(xr, stem_w, stem_b1, stem_b2, stem_b3,
      b2_1_w, b2_1_b, b3_1_w, b3_1_b, b3_2_w, b3_2_b, b4_1_w, b4_1_b)
    return out.reshape(N, cout, D, H, W)


# speedup vs baseline: 1.0598x; 1.0063x over previous
import functools

import jax
import jax.numpy as jnp
from jax.experimental import pallas as pl
from jax.experimental.pallas import tpu as pltpu

_CDT = jnp.bfloat16


def _shift_rows(a, s, fill=0.0):
    if s == 0:
        return a
    pad = jnp.full((abs(s), a.shape[1]), fill, a.dtype)
    if s > 0:
        return jnp.concatenate([a[s:], pad], axis=0)
    return jnp.concatenate([pad, a[:s]], axis=0)


def _shift_cols(a, s, fill=0.0):
    if s == 0:
        return a
    pad = jnp.full((a.shape[0], abs(s)), fill, a.dtype)
    if s > 0:
        return jnp.concatenate([a[:, s:], pad], axis=1)
    return jnp.concatenate([pad, a[:, :s]], axis=1)


def _conv3x3x3(t, w, b, masks, W, H, channel_major):
    M, C = t.shape
    mw_lo, mw_hi, mh_lo, mh_hi = masks
    tw = jnp.concatenate([
        jnp.where(mw_lo, _shift_rows(t, -1), 0).astype(t.dtype),
        t,
        jnp.where(mw_hi, _shift_rows(t, 1), 0).astype(t.dtype),
    ], axis=1)
    th = jnp.concatenate([
        jnp.where(mh_lo, _shift_rows(tw, -W), 0).astype(t.dtype),
        tw,
        jnp.where(mh_hi, _shift_rows(tw, W), 0).astype(t.dtype),
    ], axis=1)
    K9 = 9 * C
    acc = None
    for kd in range(3):
        td = _shift_rows(th, (kd - 1) * W * H)
        wk = w[kd * K9:(kd + 1) * K9, :]
        if channel_major:
            p = jax.lax.dot_general(wk, td, (((0,), (1,)), ((), ())),
                                    preferred_element_type=jnp.float32)
        else:
            p = jax.lax.dot_general(td, wk, (((1,), (0,)), ((), ())),
                                    preferred_element_type=jnp.float32)
        acc = p if acc is None else acc + p
    return jnp.maximum(acc + b, 0.0)


def _maxpool3(t, masks, W, H):
    neg = float(jnp.finfo(jnp.float32).min)
    mw_lo, mw_hi, mh_lo, mh_hi = masks

    def tap3(a, s, mlo, mhi):
        lo = jnp.where(mlo, _shift_rows(a, -s, neg), neg).astype(a.dtype)
        hi = jnp.where(mhi, _shift_rows(a, s, neg), neg).astype(a.dtype)
        return jnp.maximum(jnp.maximum(lo, a), hi)

    p = tap3(t, 1, mw_lo, mw_hi)
    p = tap3(p, W, mh_lo, mh_hi)
    return jnp.maximum(jnp.maximum(_shift_rows(p, -W * H, neg), p),
                       _shift_rows(p, W * H, neg))


def _inception_kernel(x_ref, stem_w_ref, b1_ref, b2s_ref, b3s_ref,
                      w2_ref, b2_ref, w31_ref, b31_ref,
                      w32_ref, b32_ref, w4_ref, b4_ref, o_ref,
                      *, D, H, W, c1, c2, n2, n3):
    M = D * H * W
    w1 = stem_w_ref[:, :c1].astype(_CDT)
    w23 = stem_w_ref[:, c1:].astype(_CDT)
    b1c = b1_ref[...].T
    b23 = jnp.concatenate([b2s_ref[...], b3s_ref[...]], axis=1)
    w2 = w2_ref[...].astype(_CDT)
    b2c = b2_ref[...].T
    w31 = w31_ref[...].astype(_CDT)
    b31 = b31_ref[...]
    w32 = w32_ref[...].astype(_CDT)
    b32c = b32_ref[...].T
    w4 = w4_ref[...].astype(_CDT)
    b4c = b4_ref[...].T

    x_cl = x_ref[0].astype(_CDT).T

    rhw = jax.lax.broadcasted_iota(jnp.int32, (H * W, 1), 0)
    wc = rhw % W
    hc = rhw // W
    masks = tuple(jnp.concatenate([m] * D, axis=0)
                  for m in (wc > 0, wc < W - 1, hc > 0, hc < H - 1))

    y1 = jax.lax.dot_general(w1, x_cl, (((0,), (1,)), ((), ())),
                             preferred_element_type=jnp.float32)
    o_ref[0, :c1, :] = jnp.maximum(y1 + b1c, 0.0)

    t23 = jax.lax.dot_general(x_cl, w23, (((1,), (0,)), ((), ())),
                              preferred_element_type=jnp.float32)
    t23 = jnp.maximum(t23 + b23, 0.0)
    t2 = t23[:, :c2].astype(_CDT)
    t3 = t23[:, c2:].astype(_CDT)

    y2 = _conv3x3x3(t2, w2, b2c, masks, W, H, channel_major=True)
    o_ref[0, c1:c1 + n2, :] = y2

    t3b = _conv3x3x3(t3, w31, b31, masks, W, H,
                     channel_major=False).astype(_CDT)
    y3 = _conv3x3x3(t3b, w32, b32c, masks, W, H, channel_major=True)
    o_ref[0, c1 + n2:c1 + n2 + n3, :] = y3

    pooled = _maxpool3(x_cl, masks, W, H)
    y4 = jax.lax.dot_general(w4, pooled, (((0,), (1,)), ((), ())),
                             preferred_element_type=jnp.float32)
    o_ref[0, c1 + n2 + n3:, :] = jnp.maximum(y4 + b4c, 0.0)


def kernel(x, stem_w, stem_b1, stem_b2, stem_b3,
           b2_1_w, b2_1_b, b3_1_w, b3_1_b, b3_2_w, b3_2_b,
           b4_1_w, b4_1_b):
    N, Cin, D, H, W = x.shape
    M = D * H * W
    c1 = stem_b1.shape[1]
    c2 = stem_b2.shape[1]
    n2 = b2_1_w.shape[1]
    n3 = b3_2_w.shape[1]
    n4 = b4_1_w.shape[1]
    cout = c1 + n2 + n3 + n4

    xr = x.reshape(N, Cin, M)

    kfn = functools.partial(_inception_kernel, D=D, H=H, W=W,
                            c1=c1, c2=c2, n2=n2, n3=n3)

    def full(a):
        nd = len(a.shape)
        return pl.BlockSpec(a.shape, lambda n, _nd=nd: (0,) * _nd)

    out = pl.pallas_call(
        kfn,
        out_shape=jax.ShapeDtypeStruct((N, cout, M), jnp.float32),
        grid=(N,),
        in_specs=[
            pl.BlockSpec((1, Cin, M), lambda n: (n, 0, 0)),
            full(stem_w), full(stem_b1), full(stem_b2), full(stem_b3),
            full(b2_1_w), full(b2_1_b), full(b3_1_w), full(b3_1_b),
            full(b3_2_w), full(b3_2_b), full(b4_1_w), full(b4_1_b),
        ],
        out_specs=pl.BlockSpec((1, cout, M), lambda n: (n, 0, 0)),
        compiler_params=pltpu.CompilerParams(
            dimension_semantics=("parallel",)),
    )(xr, stem_w, stem_b1, stem_b2, stem_b3,
      b2_1_w, b2_1_b, b3_1_w, b3_1_b, b3_2_w, b3_2_b, b4_1_w, b4_1_b)
    return out.reshape(N, cout, D, H, W)
